# Initial kernel scaffold; baseline (speedup 1.0000x reference)
import jax
import jax.numpy as jnp
from jax.experimental import pallas as pl


def kernel(z, embedding):
    raise NotImplementedError("write your pallas kernel here")


# speedup vs baseline: 1.0964x; 1.0964x over previous
import functools

import jax
import jax.numpy as jnp
from jax.experimental import pallas as pl
from jax.experimental.pallas import tpu as pltpu
from jax.experimental.pallas import tpu_sc as plsc

_N_E = 8192
_D = 256
_BR = 2048
_BC = 2048


def _dist_argmin_body(z_ref, zsq_ref, e_ref, idx_ref, minv_ref, mini_ref):
    c = pl.program_id(1)
    zb = z_ref[...]
    eb = e_ref[...]
    esq = jnp.sum(eb * eb, axis=1)
    zsq = zsq_ref[...][:, 0]
    m = jax.lax.dot_general(
        zb, eb, (((1,), (1,)), ((), ())),
        preferred_element_type=jnp.float32)
    d = (zsq[:, None] + esq[None, :]) - 2.0 * m
    bmin = jnp.min(d, axis=1)
    ii = jax.lax.broadcasted_iota(jnp.int32, d.shape, 1)
    barg = jnp.min(jnp.where(d == bmin[:, None], ii, jnp.int32(2**30)),
                   axis=1) + c * _BC

    @pl.when(c == 0)
    def _():
        minv_ref[...] = bmin.astype(jnp.bfloat16).astype(jnp.float32)[:, None]
        mini_ref[...] = barg[:, None]

    @pl.when(c > 0)
    def _():
        prev = minv_ref[...][:, 0]
        pidx = mini_ref[...][:, 0]
        upd = bmin < prev
        newv = jnp.where(upd, bmin, prev)
        minv_ref[...] = newv.astype(jnp.bfloat16).astype(jnp.float32)[:, None]
        mini_ref[...] = jnp.where(upd, barg, pidx)[:, None]

    @pl.when(c == (_N_E // _BC) - 1)
    def _():
        idx_ref[...] = mini_ref[...][:, 0][None, None, :]


def _argmin_indices(z_flat, zsq, embedding):
    n = z_flat.shape[0]
    grid = (n // _BR, _N_E // _BC)
    out = pl.pallas_call(
        _dist_argmin_body,
        grid=grid,
        in_specs=[
            pl.BlockSpec((_BR, _D), lambda r, c: (r, 0)),
            pl.BlockSpec((_BR, 1), lambda r, c: (r, 0)),
            pl.BlockSpec((_BC, _D), lambda r, c: (c, 0)),
        ],
        out_specs=pl.BlockSpec((1, 1, _BR), lambda r, c: (r, 0, 0)),
        out_shape=jax.ShapeDtypeStruct((n // _BR, 1, _BR), jnp.int32),
        scratch_shapes=[
            pltpu.VMEM((_BR, 1), jnp.float32),
            pltpu.VMEM((_BR, 1), jnp.int32),
        ],
    )(z_flat, zsq, embedding)
    return out.reshape(n)


def _gather_rows(embedding, indices):
    n = indices.shape[0]
    window = 128
    mesh = plsc.VectorSubcoreMesh(core_axis_name="core",
                                  subcore_axis_name="subcore")
    idx2d = indices.reshape(1, n)

    @functools.partial(
        pl.kernel,
        out_type=jax.ShapeDtypeStruct((n, _D), embedding.dtype),
        mesh=mesh,
    )
    def _sc_gather(e_hbm, i_hbm, o_hbm):
        def body(i_vmem, o_vmem):
            pltpu.sync_copy(e_hbm.at[i_vmem.at[0]], o_vmem)

        pltpu.emit_pipeline(
            body,
            grid=(n // window,),
            in_specs=[pl.BlockSpec((1, window), index_map=lambda i: (0, i))],
            out_specs=[pl.BlockSpec((window, _D), index_map=lambda i: (i, 0))],
            core_axis_name=("core", "subcore"),
            dimension_semantics=(pltpu.PARALLEL,),
        )(i_hbm, o_hbm)

    return _sc_gather(embedding, idx2d)


def _finalize_body(z_ref, q_ref, out_ref, lsum_ref):
    z = z_ref[...]
    q = q_ref[...]
    diff = q - z
    out_ref[...] = z + diff
    lsum_ref[0, 0] = jnp.sum(diff * diff)


def _finalize(z_flat, zq_flat):
    n = z_flat.shape[0]
    out, lsum = pl.pallas_call(
        _finalize_body,
        in_specs=[
            pl.BlockSpec((n, _D), lambda: (0, 0)),
            pl.BlockSpec((n, _D), lambda: (0, 0)),
        ],
        out_specs=[
            pl.BlockSpec((n, _D), lambda: (0, 0)),
            pl.BlockSpec(memory_space=pltpu.SMEM),
        ],
        out_shape=[
            jax.ShapeDtypeStruct((n, _D), jnp.float32),
            jax.ShapeDtypeStruct((1, 1), jnp.float32),
        ],
    )(z_flat, zq_flat)
    count = jnp.float32(n * _D)
    loss = lsum[0, 0] / count + lsum[0, 0] / count
    return out, loss


@jax.jit
def kernel(z, embedding):
    z_flat = jnp.reshape(z, (-1, _D))
    zsq = jnp.sum(z_flat ** 2, axis=1, keepdims=True)
    indices = _argmin_indices(z_flat, zsq, embedding)
    zq_flat = _gather_rows(embedding, indices)
    out_flat, loss = _finalize(z_flat, zq_flat)
    return jnp.reshape(out_flat, z.shape), loss, indices


# speedup vs baseline: 1.2262x; 1.1183x over previous
import functools

import jax
import jax.numpy as jnp
from jax.experimental import pallas as pl
from jax.experimental.pallas import tpu as pltpu
from jax.experimental.pallas import tpu_sc as plsc

_N_E = 8192
_D = 256
_BR = 2048
_BC = 2048


def _dist_argmin_body(z_ref, zsq_ref, e_ref, idx_ref, minv_ref, mini_ref):
    c = pl.program_id(1)
    eb = e_ref[...]
    esq = jnp.sum(eb * eb, axis=1)
    zsq = zsq_ref[...][:, 0]
    m2 = jax.lax.dot_general(
        z_ref[...] * -2.0, eb, (((1,), (1,)), ((), ())),
        preferred_element_type=jnp.float32)
    s = zsq[:, None] + esq[None, :]
    lanes = 128
    ii0 = jax.lax.broadcasted_iota(jnp.int32, (_BR, lanes), 1)
    av = s[:, :lanes] + m2[:, :lanes]
    ai = ii0
    for k in range(1, _BC // lanes):
        nd = s[:, k * lanes:(k + 1) * lanes] + m2[:, k * lanes:(k + 1) * lanes]
        lt = nd < av
        av = jnp.where(lt, nd, av)
        ai = jnp.where(lt, ii0 + (k * lanes), ai)
    bmin = jnp.min(av, axis=1)
    barg = jnp.min(jnp.where(av == bmin[:, None], ai, jnp.int32(2 ** 30)),
                   axis=1) + c * _BC

    @pl.when(c == 0)
    def _():
        minv_ref[...] = bmin.astype(jnp.bfloat16).astype(jnp.float32)[:, None]
        mini_ref[...] = barg[:, None]

    @pl.when(c > 0)
    def _():
        prev = minv_ref[...][:, 0]
        pidx = mini_ref[...][:, 0]
        upd = bmin < prev
        newv = jnp.where(upd, bmin, prev)
        minv_ref[...] = newv.astype(jnp.bfloat16).astype(jnp.float32)[:, None]
        mini_ref[...] = jnp.where(upd, barg, pidx)[:, None]

    @pl.when(c == (_N_E // _BC) - 1)
    def _():
        idx_ref[...] = mini_ref[...][:, 0][None, None, :]


def _argmin_indices(z_flat, zsq, embedding):
    n = z_flat.shape[0]
    grid = (n // _BR, _N_E // _BC)
    out = pl.pallas_call(
        _dist_argmin_body,
        grid=grid,
        in_specs=[
            pl.BlockSpec((_BR, _D), lambda r, c: (r, 0)),
            pl.BlockSpec((_BR, 1), lambda r, c: (r, 0)),
            pl.BlockSpec((_BC, _D), lambda r, c: (c, 0)),
        ],
        out_specs=pl.BlockSpec((1, 1, _BR), lambda r, c: (r, 0, 0)),
        out_shape=jax.ShapeDtypeStruct((n // _BR, 1, _BR), jnp.int32),
        scratch_shapes=[
            pltpu.VMEM((_BR, 1), jnp.float32),
            pltpu.VMEM((_BR, 1), jnp.int32),
        ],
    )(z_flat, zsq, embedding)
    return out.reshape(n)


def _gather_rows(embedding, indices):
    n = indices.shape[0]
    window = 128
    mesh = plsc.VectorSubcoreMesh(core_axis_name="core",
                                  subcore_axis_name="subcore")
    idx2d = indices.reshape(1, n)

    @functools.partial(
        pl.kernel,
        out_type=jax.ShapeDtypeStruct((n, _D), embedding.dtype),
        mesh=mesh,
    )
    def _sc_gather(e_hbm, i_hbm, o_hbm):
        def body(i_vmem, o_vmem):
            pltpu.sync_copy(e_hbm.at[i_vmem.at[0]], o_vmem)

        pltpu.emit_pipeline(
            body,
            grid=(n // window,),
            in_specs=[pl.BlockSpec((1, window), index_map=lambda i: (0, i))],
            out_specs=[pl.BlockSpec((window, _D), index_map=lambda i: (i, 0))],
            core_axis_name=("core", "subcore"),
            dimension_semantics=(pltpu.PARALLEL,),
        )(i_hbm, o_hbm)

    return _sc_gather(embedding, idx2d)


def _finalize_body(z_ref, q_ref, out_ref, lsum_ref):
    z = z_ref[...]
    q = q_ref[...]
    diff = q - z
    out_ref[...] = z + diff
    lsum_ref[0, 0] = jnp.sum(diff * diff)


def _finalize(z_flat, zq_flat):
    n = z_flat.shape[0]
    out, lsum = pl.pallas_call(
        _finalize_body,
        in_specs=[
            pl.BlockSpec((n, _D), lambda: (0, 0)),
            pl.BlockSpec((n, _D), lambda: (0, 0)),
        ],
        out_specs=[
            pl.BlockSpec((n, _D), lambda: (0, 0)),
            pl.BlockSpec(memory_space=pltpu.SMEM),
        ],
        out_shape=[
            jax.ShapeDtypeStruct((n, _D), jnp.float32),
            jax.ShapeDtypeStruct((1, 1), jnp.float32),
        ],
    )(z_flat, zq_flat)
    count = jnp.float32(n * _D)
    loss = lsum[0, 0] / count + lsum[0, 0] / count
    return out, loss


@jax.jit
def kernel(z, embedding):
    z_flat = jnp.reshape(z, (-1, _D))
    zsq = jnp.sum(z_flat ** 2, axis=1, keepdims=True)
    indices = _argmin_indices(z_flat, zsq, embedding)
    zq_flat = _gather_rows(embedding, indices)
    out_flat, loss = _finalize(z_flat, zq_flat)
    return jnp.reshape(out_flat, z.shape), loss, indices


# speedup vs baseline: 1.2322x; 1.0049x over previous
import functools

import jax
import jax.numpy as jnp
from jax.experimental import pallas as pl
from jax.experimental.pallas import tpu as pltpu
from jax.experimental.pallas import tpu_sc as plsc

_N_E = 8192
_D = 256
_BR = 2048
_BC = 2048


def _dist_argmin_body(z_ref, zsq_ref, e_ref, idx_ref, minv_ref, mini_ref):
    c = pl.program_id(1)
    eb = e_ref[...]
    esq = jnp.sum(eb * eb, axis=1)
    zsq = zsq_ref[...][:, 0]
    m2 = jax.lax.dot_general(
        z_ref[...] * -2.0, eb, (((1,), (1,)), ((), ())),
        preferred_element_type=jnp.float32)
    lanes = 128
    zsqc = zsq[:, None]
    ii0 = jax.lax.broadcasted_iota(jnp.int32, (_BR, lanes), 1)
    av = (zsqc + esq[None, :lanes]) + m2[:, :lanes]
    ai = ii0
    for k in range(1, _BC // lanes):
        nd = (zsqc + esq[None, k * lanes:(k + 1) * lanes]
              ) + m2[:, k * lanes:(k + 1) * lanes]
        lt = nd < av
        av = jnp.where(lt, nd, av)
        ai = jnp.where(lt, ii0 + (k * lanes), ai)
    bmin = jnp.min(av, axis=1)
    barg = jnp.min(jnp.where(av == bmin[:, None], ai, jnp.int32(2 ** 30)),
                   axis=1) + c * _BC

    @pl.when(c == 0)
    def _():
        minv_ref[...] = bmin.astype(jnp.bfloat16).astype(jnp.float32)[:, None]
        mini_ref[...] = barg[:, None]

    @pl.when(c > 0)
    def _():
        prev = minv_ref[...][:, 0]
        pidx = mini_ref[...][:, 0]
        upd = bmin < prev
        newv = jnp.where(upd, bmin, prev)
        minv_ref[...] = newv.astype(jnp.bfloat16).astype(jnp.float32)[:, None]
        mini_ref[...] = jnp.where(upd, barg, pidx)[:, None]

    @pl.when(c == (_N_E // _BC) - 1)
    def _():
        idx_ref[...] = mini_ref[...][:, 0][None, None, :]


def _argmin_indices(z_flat, zsq, embedding):
    n = z_flat.shape[0]
    grid = (n // _BR, _N_E // _BC)
    out = pl.pallas_call(
        _dist_argmin_body,
        grid=grid,
        in_specs=[
            pl.BlockSpec((_BR, _D), lambda r, c: (r, 0)),
            pl.BlockSpec((_BR, 1), lambda r, c: (r, 0)),
            pl.BlockSpec((_BC, _D), lambda r, c: (c, 0)),
        ],
        out_specs=pl.BlockSpec((1, 1, _BR), lambda r, c: (r, 0, 0)),
        out_shape=jax.ShapeDtypeStruct((n // _BR, 1, _BR), jnp.int32),
        scratch_shapes=[
            pltpu.VMEM((_BR, 1), jnp.float32),
            pltpu.VMEM((_BR, 1), jnp.int32),
        ],
    )(z_flat, zsq, embedding)
    return out.reshape(n)


def _gather_rows(embedding, indices):
    n = indices.shape[0]
    window = 128
    mesh = plsc.VectorSubcoreMesh(core_axis_name="core",
                                  subcore_axis_name="subcore")
    idx2d = indices.reshape(1, n)

    @functools.partial(
        pl.kernel,
        out_type=jax.ShapeDtypeStruct((n, _D), embedding.dtype),
        mesh=mesh,
    )
    def _sc_gather(e_hbm, i_hbm, o_hbm):
        def body(i_vmem, o_vmem):
            pltpu.sync_copy(e_hbm.at[i_vmem.at[0]], o_vmem)

        pltpu.emit_pipeline(
            body,
            grid=(n // window,),
            in_specs=[pl.BlockSpec((1, window), index_map=lambda i: (0, i))],
            out_specs=[pl.BlockSpec((window, _D), index_map=lambda i: (i, 0))],
            core_axis_name=("core", "subcore"),
            dimension_semantics=(pltpu.PARALLEL,),
        )(i_hbm, o_hbm)

    return _sc_gather(embedding, idx2d)


def _finalize_body(z_ref, q_ref, out_ref, lsum_ref):
    z = z_ref[...]
    q = q_ref[...]
    diff = q - z
    out_ref[...] = z + diff
    lsum_ref[0, 0] = jnp.sum(diff * diff)


def _finalize(z_flat, zq_flat):
    n = z_flat.shape[0]
    out, lsum = pl.pallas_call(
        _finalize_body,
        in_specs=[
            pl.BlockSpec((n, _D), lambda: (0, 0)),
            pl.BlockSpec((n, _D), lambda: (0, 0)),
        ],
        out_specs=[
            pl.BlockSpec((n, _D), lambda: (0, 0)),
            pl.BlockSpec(memory_space=pltpu.SMEM),
        ],
        out_shape=[
            jax.ShapeDtypeStruct((n, _D), jnp.float32),
            jax.ShapeDtypeStruct((1, 1), jnp.float32),
        ],
    )(z_flat, zq_flat)
    count = jnp.float32(n * _D)
    loss = lsum[0, 0] / count + lsum[0, 0] / count
    return out, loss


@jax.jit
def kernel(z, embedding):
    z_flat = jnp.reshape(z, (-1, _D))
    zsq = jnp.sum(z_flat ** 2, axis=1, keepdims=True)
    indices = _argmin_indices(z_flat, zsq, embedding)
    zq_flat = _gather_rows(embedding, indices)
    out_flat, loss = _finalize(z_flat, zq_flat)
    return jnp.reshape(out_flat, z.shape), loss, indices
